# Initial kernel scaffold; baseline (speedup 1.0000x reference)
import jax
import jax.numpy as jnp
from jax.experimental import pallas as pl


def kernel(adj, feats, table, W1, b1, Wfc, bfc):
    raise NotImplementedError("write your pallas kernel here")


# speedup vs baseline: 1.7649x; 1.7649x over previous
import functools

import jax
import jax.numpy as jnp
from jax import lax
from jax.experimental import pallas as pl
from jax.experimental.pallas import tpu as pltpu
from jax.experimental.pallas import tpu_sc as plsc

V = 100000
D = 128
B = 16384
NC = 2
NS = 16
NW = NC * NS
BPW = B // NW

_mesh = plsc.VectorSubcoreMesh(core_axis_name="c", subcore_axis_name="s")


@functools.partial(
    pl.kernel,
    out_type=jax.ShapeDtypeStruct((B, D), jnp.float32),
    mesh=_mesh,
    scratch_types=[
        pltpu.VMEM((BPW,), jnp.int32),
        pltpu.VMEM((BPW, D), jnp.float32),
        pltpu.SemaphoreType.DMA,
    ],
)
def _sc_gather(table_hbm, idx_hbm, out_hbm, idx_v, rows_v, sem):
    wid = lax.axis_index("s") * NC + lax.axis_index("c")
    base = wid * BPW
    pltpu.sync_copy(idx_hbm.at[pl.ds(base, BPW)], idx_v)
    pltpu.async_copy(table_hbm.at[idx_v], rows_v, sem).wait()
    pltpu.sync_copy(rows_v, out_hbm.at[pl.ds(base, BPW)])


def _tc_mlp(emb_ref, w1_ref, featst_ref, params_ref, out_ref):
    s = jnp.sum(emb_ref[...] * w1_ref[...], axis=1)
    t = jnp.tanh(s + params_ref[0, 0])
    out_ref[...] = (
        featst_ref[0, :] * params_ref[0, 1]
        + featst_ref[1, :] * params_ref[0, 2]
        + t * params_ref[0, 3]
        + params_ref[0, 4]
    )


def kernel(adj, feats, table, W1, b1, Wfc, bfc):
    emb = _sc_gather(table, adj.astype(jnp.int32))
    w1row = W1.reshape(1, D)
    featst = feats.T
    params = jnp.stack(
        [b1[0], Wfc[0, 0], Wfc[1, 0], Wfc[2, 0], bfc[0]]
    ).reshape(1, 5)
    ret = pl.pallas_call(
        _tc_mlp,
        out_shape=jax.ShapeDtypeStruct((B,), jnp.float32),
        in_specs=[
            pl.BlockSpec(memory_space=pltpu.VMEM),
            pl.BlockSpec(memory_space=pltpu.VMEM),
            pl.BlockSpec(memory_space=pltpu.VMEM),
            pl.BlockSpec(memory_space=pltpu.SMEM),
        ],
        out_specs=pl.BlockSpec(memory_space=pltpu.VMEM),
    )(emb, w1row, featst, params)
    return ret.reshape(B, 1)


# speedup vs baseline: 1.9808x; 1.1223x over previous
import functools

import jax
import jax.numpy as jnp
from jax import lax
from jax.experimental import pallas as pl
from jax.experimental.pallas import tpu as pltpu
from jax.experimental.pallas import tpu_sc as plsc

V = 100000
D = 128
B = 16384
NC = 2
NS = 16
NW = NC * NS
BPW = B // NW
HALF = BPW // 2
NCH = D // 16

_mesh = plsc.VectorSubcoreMesh(core_axis_name="c", subcore_axis_name="s")


@functools.partial(
    pl.kernel,
    out_type=jax.ShapeDtypeStruct((B,), jnp.float32),
    mesh=_mesh,
    scratch_types=[
        pltpu.VMEM((BPW,), jnp.int32),
        pltpu.VMEM((HALF, D), jnp.float32),
        pltpu.VMEM((HALF, D), jnp.float32),
        pltpu.VMEM((D,), jnp.float32),
        pltpu.VMEM((BPW,), jnp.float32),
        pltpu.VMEM((BPW,), jnp.float32),
        pltpu.VMEM((D,), jnp.float32),
        pltpu.VMEM((BPW,), jnp.float32),
        pltpu.SemaphoreType.DMA,
        pltpu.SemaphoreType.DMA,
    ],
)
def _sc_fused(table_hbm, idx_hbm, w1_hbm, fp_hbm, consts_hbm, out_hbm,
              idx_v, rows0_v, rows1_v, w1_v, f0_v, f1_v, consts_v,
              out_v, sem0, sem1):
    wid = lax.axis_index("s") * NC + lax.axis_index("c")
    base = wid * BPW

    pltpu.sync_copy(idx_hbm.at[pl.ds(base, BPW)], idx_v)
    cp0 = pltpu.async_copy(table_hbm.at[idx_v.at[pl.ds(0, HALF)]], rows0_v,
                           sem0)
    cp1 = pltpu.async_copy(table_hbm.at[idx_v.at[pl.ds(HALF, HALF)]], rows1_v,
                           sem1)
    pltpu.sync_copy(w1_hbm, w1_v)
    pltpu.sync_copy(fp_hbm.at[pl.ds(base, BPW)], f0_v)
    pltpu.sync_copy(fp_hbm.at[pl.ds(B + base, BPW)], f1_v)
    pltpu.sync_copy(consts_hbm, consts_v)

    w1c = [w1_v[pl.ds(16 * c, 16)] for c in range(NCH)]
    c_b1 = consts_v[pl.ds(0, 16)]
    c_w0 = consts_v[pl.ds(16, 16)]
    c_w1 = consts_v[pl.ds(32, 16)]
    c_w2 = consts_v[pl.ds(48, 16)]
    c_bfc = consts_v[pl.ds(64, 16)]
    lanes = lax.iota(jnp.int32, 16)
    one = jnp.full((16,), 1.0, jnp.float32)
    two = jnp.full((16,), 2.0, jnp.float32)

    p8 = lanes ^ 8
    p4 = lanes ^ 4
    p2 = lanes ^ 2
    p1 = lanes ^ 1

    def hsum(v):
        v = v + v.at[p8].get(mode="promise_in_bounds")
        v = v + v.at[p4].get(mode="promise_in_bounds")
        v = v + v.at[p2].get(mode="promise_in_bounds")
        v = v + v.at[p1].get(mode="promise_in_bounds")
        return v

    def do_half(rows_ref, off):
        @plsc.parallel_loop(0, HALF // 16, 1, unroll=1)
        def _(g):
            y = jnp.zeros((16,), jnp.float32)
            for j in range(16):
                r = 16 * g + j
                acc = rows_ref[r, pl.ds(0, 16)] * w1c[0]
                for c in range(1, NCH):
                    acc = acc + rows_ref[r, pl.ds(16 * c, 16)] * w1c[c]
                y = jnp.where(lanes == j, hsum(acc), y)
            x = y + c_b1
            ax = jnp.abs(x)
            e = jnp.exp(two * ax)
            t = one - two / (e + one)
            t = jnp.where(x < 0.0, -t, t)
            b16 = off + 16 * g
            r16 = (f0_v[pl.ds(b16, 16)] * c_w0
                   + f1_v[pl.ds(b16, 16)] * c_w1
                   + t * c_w2 + c_bfc)
            out_v[pl.ds(b16, 16)] = r16

    cp0.wait()
    do_half(rows0_v, 0)
    cp1.wait()
    do_half(rows1_v, HALF)

    pltpu.sync_copy(out_v, out_hbm.at[pl.ds(base, BPW)])


def kernel(adj, feats, table, W1, b1, Wfc, bfc):
    idx = adj.astype(jnp.int32)
    w1flat = W1.reshape(D)
    fpack = feats.T.reshape(2 * B)
    consts = jnp.concatenate([
        jnp.full((16,), b1[0], jnp.float32),
        jnp.full((16,), Wfc[0, 0], jnp.float32),
        jnp.full((16,), Wfc[1, 0], jnp.float32),
        jnp.full((16,), Wfc[2, 0], jnp.float32),
        jnp.full((16,), bfc[0], jnp.float32),
        jnp.zeros((D - 80,), jnp.float32),
    ])
    ret = _sc_fused(table, idx, w1flat, fpack, consts)
    return ret.reshape(B, 1)
